# baseline (speedup 1.0000x reference)
import jax
import jax.numpy as jnp
from jax import lax
from jax.experimental import pallas as pl
from jax.experimental.pallas import tpu as pltpu
from jax.experimental.pallas import tpu_sc as plsc

E = 160000
USERS = 5000
ITEMS = 5000
D = 256
NR = 5

NC = 2
NS = 16
NW = NC * NS
EP = 163840
EW = EP // NW
CH = 128
NCH = EW // CH
UP = 5120
USC = UP // NS
C5 = 64

_mesh = lambda: plsc.VectorSubcoreMesh(
    core_axis_name="c", subcore_axis_name="s", num_cores=NC, num_subcores=NS)
_SC_PARAMS = pltpu.CompilerParams(needs_layout_passes=False)


def _leaky(x):
  return jnp.where(x >= 0, x, 0.01 * x)


def _tables_body(item_ref, user_ref, rating_ref, w1i_ref, w1r_ref, b1_ref,
                 w2_ref, b2_ref, a1x_ref, a1u_ref, a1b_ref,
                 xia_ref, xa1_ref, up1_ref):
  r = pl.program_id(0)
  rp1_all = jnp.dot(rating_ref[...], w1r_ref[...],
                    preferred_element_type=jnp.float32) + b1_ref[...]
  onehot = (lax.broadcasted_iota(jnp.int32, (1, 8), 1) == r).astype(jnp.float32)
  rp1 = jnp.dot(onehot, rp1_all, preferred_element_type=jnp.float32)
  ip1 = jnp.dot(item_ref[...], w1i_ref[...], preferred_element_type=jnp.float32)
  h1 = _leaky(ip1 + rp1)
  xia = _leaky(jnp.dot(h1, w2_ref[...], preferred_element_type=jnp.float32)
               + b2_ref[...])
  xia_ref[...] = xia[None]
  xa1_ref[...] = jnp.dot(xia, a1x_ref[...],
                         preferred_element_type=jnp.float32)[None]
  up1_ref[...] = jnp.dot(user_ref[...], a1u_ref[...],
                         preferred_element_type=jnp.float32) + a1b_ref[...]


def _build_tables(item_feat, user_feat, rating_pad, w1iT, w1rT, b1, w2T, b2,
                  a1xT, a1uT, a1b):
  nb = 5
  full = lambda *dims: pl.BlockSpec(dims, lambda r, b: tuple(0 for _ in dims))
  return pl.pallas_call(
      _tables_body,
      grid=(NR, nb),
      in_specs=[
          pl.BlockSpec((1000, D), lambda r, b: (b, 0)),
          pl.BlockSpec((1000, D), lambda r, b: (b, 0)),
          full(8, D), full(D, D), full(D, D), full(1, D),
          full(D, D), full(1, D), full(D, D), full(D, D), full(1, D),
      ],
      out_specs=[
          pl.BlockSpec((1, 1000, D), lambda r, b: (r, b, 0)),
          pl.BlockSpec((1, 1000, D), lambda r, b: (r, b, 0)),
          pl.BlockSpec((1000, D), lambda r, b: (b, 0)),
      ],
      out_shape=[
          jax.ShapeDtypeStruct((NR, ITEMS, D), jnp.float32),
          jax.ShapeDtypeStruct((NR, ITEMS, D), jnp.float32),
          jax.ShapeDtypeStruct((USERS, D), jnp.float32),
      ],
  )(item_feat, user_feat, rating_pad, w1iT, w1rT, b1, w2T, b2, a1xT, a1uT, a1b)


def _gather_body(xa1_hbm, up1_hbm, i_hbm, r_hbm, u_hbm, g_hbm,
                 ibuf, rbuf, ubuf, pbuf, rowsA, rowsB, semA, semB):
  cid = lax.axis_index("c")
  sid = lax.axis_index("s")
  wid = sid * NC + cid
  base = wid * EW
  pltpu.sync_copy(i_hbm.at[pl.ds(base, EW)], ibuf)
  pltpu.sync_copy(r_hbm.at[pl.ds(base, EW)], rbuf)
  pltpu.sync_copy(u_hbm.at[pl.ds(base, EW)], ubuf)

  def chunk(c, carry):
    off = c * CH
    for g in range(CH // 16):
      o = off + g * 16
      iv = ibuf[pl.ds(o, 16)]
      rv = rbuf[pl.ds(o, 16)]
      pbuf[pl.ds(g * 16, 16)] = rv * ITEMS + iv
    cpA = pltpu.async_copy(xa1_hbm.at[pbuf], rowsA, semA)
    cpB = pltpu.async_copy(up1_hbm.at[ubuf.at[pl.ds(off, CH)]], rowsB, semB)
    cpA.wait()
    cpB.wait()

    def row(j, rc):
      for cc in range(D // 16):
        x = rowsA[j, pl.ds(cc * 16, 16)] + rowsB[j, pl.ds(cc * 16, 16)]
        rowsA[j, pl.ds(cc * 16, 16)] = jnp.maximum(x, 0.0)
      return rc
    lax.fori_loop(0, CH, row, 0)
    pltpu.sync_copy(rowsA, g_hbm.at[pl.ds(base + off, CH)])
    return carry
  lax.fori_loop(0, NCH, chunk, 0)


def _run_gather(xa1_flat, up1, ip, rp, up):
  k = pl.kernel(
      _gather_body,
      out_type=jax.ShapeDtypeStruct((EP, D), jnp.float32),
      mesh=_mesh(),
      compiler_params=_SC_PARAMS,
      scratch_types=[
          pltpu.VMEM((EW,), jnp.int32),
          pltpu.VMEM((EW,), jnp.int32),
          pltpu.VMEM((EW,), jnp.int32),
          pltpu.VMEM((CH,), jnp.int32),
          pltpu.VMEM((CH, D), jnp.float32),
          pltpu.VMEM((CH, D), jnp.float32),
          pltpu.SemaphoreType.DMA,
          pltpu.SemaphoreType.DMA,
      ],
  )
  return k(xa1_flat, up1, ip, rp, up)


def _score_body(g_ref, a2_ref, a2b_ref, a3_ref, out_ref):
  a2 = jnp.maximum(
      jnp.dot(g_ref[...], a2_ref[...], preferred_element_type=jnp.float32)
      + a2b_ref[...], 0.0)
  s = jnp.sum(a2 * a3_ref[...], axis=1)
  out_ref[...] = s.reshape(1, 1, 1280)


def _run_score(g, a2T, a2b, a3):
  nb = EP // 1280
  s3 = pl.pallas_call(
      _score_body,
      grid=(nb,),
      in_specs=[
          pl.BlockSpec((1280, D), lambda b: (b, 0)),
          pl.BlockSpec((D, D), lambda b: (0, 0)),
          pl.BlockSpec((1, D), lambda b: (0, 0)),
          pl.BlockSpec((1, D), lambda b: (0, 0)),
      ],
      out_specs=pl.BlockSpec((1, 1, 1280), lambda b: (b, 0, 0)),
      out_shape=jax.ShapeDtypeStruct((nb, 1, 1280), jnp.float32),
  )(g, a2T, a2b, a3)
  return s3.reshape(EP)


def _segmax_body(s_hbm, u_hbm, mp_hbm, acc, sbuf, ubuf, mrow):
  cid = lax.axis_index("c")
  sid = lax.axis_index("s")
  wid = sid * NC + cid
  base = wid * EW

  def init(k, c):
    acc[pl.ds(k * 16, 16)] = jnp.full((16,), -1e30, jnp.float32)
    return c
  lax.fori_loop(0, (16 * UP) // 16, init, 0)

  pltpu.sync_copy(s_hbm.at[pl.ds(base, EW)], sbuf)
  pltpu.sync_copy(u_hbm.at[pl.ds(base, EW)], ubuf)
  lanes = lax.iota(jnp.int32, 16) * UP

  def grp(g, c):
    uv = ubuf[pl.ds(g * 16, 16)]
    sv = sbuf[pl.ds(g * 16, 16)]
    idx = lanes + uv
    cur = plsc.load_gather(acc, [idx])
    plsc.store_scatter(acc, [idx], jnp.maximum(cur, sv))
    return c
  lax.fori_loop(0, EW // 16, grp, 0)

  def red(t, c):
    b = t * 16
    m = acc[pl.ds(b, 16)]
    for l in range(1, 16):
      m = jnp.maximum(m, acc[pl.ds(l * UP + b, 16)])
    mrow[pl.ds(b, 16)] = m
    return c
  lax.fori_loop(0, UP // 16, red, 0)
  pltpu.sync_copy(mrow, mp_hbm.at[wid])


def _run_segmax(s, up):
  k = pl.kernel(
      _segmax_body,
      out_type=jax.ShapeDtypeStruct((NW, UP), jnp.float32),
      mesh=_mesh(),
      compiler_params=_SC_PARAMS,
      scratch_types=[
          pltpu.VMEM((16 * UP,), jnp.float32),
          pltpu.VMEM((EW,), jnp.float32),
          pltpu.VMEM((EW,), jnp.int32),
          pltpu.VMEM((UP,), jnp.float32),
      ],
  )
  return k(s, up)


def _combine_partials(src_hbm, mtmp, dst, is_max, rows_per_dma):
  neutral = -1e30 if is_max else 0.0

  def init(t, c):
    dst[pl.ds(t * 16, 16)] = jnp.full((16,), neutral, jnp.float32)
    return c
  lax.fori_loop(0, UP // 16, init, 0)
  for blk in range(NW // rows_per_dma):
    pltpu.sync_copy(src_hbm.at[pl.ds(blk * rows_per_dma, rows_per_dma)], mtmp)

    def red(t, c):
      b = t * 16
      v = dst[pl.ds(b, 16)]
      for r in range(rows_per_dma):
        w = mtmp[r, pl.ds(b, 16)]
        v = jnp.maximum(v, w) if is_max else v + w
      dst[pl.ds(b, 16)] = v
      return c
    lax.fori_loop(0, UP // 16, red, 0)


def _segsum_body(s_hbm, u_hbm, mp_hbm, ssp_hbm, acc, sbuf, ubuf, mvec, mtmp,
                 srow):
  cid = lax.axis_index("c")
  sid = lax.axis_index("s")
  wid = sid * NC + cid
  base = wid * EW

  _combine_partials(mp_hbm, mtmp, mvec, True, 4)

  def init(k, c):
    acc[pl.ds(k * 16, 16)] = jnp.zeros((16,), jnp.float32)
    return c
  lax.fori_loop(0, (16 * UP) // 16, init, 0)

  pltpu.sync_copy(s_hbm.at[pl.ds(base, EW)], sbuf)
  pltpu.sync_copy(u_hbm.at[pl.ds(base, EW)], ubuf)
  lanes = lax.iota(jnp.int32, 16) * UP

  def grp(g, c):
    uv = ubuf[pl.ds(g * 16, 16)]
    sv = sbuf[pl.ds(g * 16, 16)]
    mv = plsc.load_gather(mvec, [uv])
    ex = jnp.exp(sv - mv)
    plsc.addupdate_scatter(acc, [lanes + uv], ex)
    return c
  lax.fori_loop(0, EW // 16, grp, 0)

  def red(t, c):
    b = t * 16
    v = acc[pl.ds(b, 16)]
    for l in range(1, 16):
      v = v + acc[pl.ds(l * UP + b, 16)]
    srow[pl.ds(b, 16)] = v
    return c
  lax.fori_loop(0, UP // 16, red, 0)
  pltpu.sync_copy(srow, ssp_hbm.at[wid])


def _run_segsum(s, up, mp):
  k = pl.kernel(
      _segsum_body,
      out_type=jax.ShapeDtypeStruct((NW, UP), jnp.float32),
      mesh=_mesh(),
      compiler_params=_SC_PARAMS,
      scratch_types=[
          pltpu.VMEM((16 * UP,), jnp.float32),
          pltpu.VMEM((EW,), jnp.float32),
          pltpu.VMEM((EW,), jnp.int32),
          pltpu.VMEM((UP,), jnp.float32),
          pltpu.VMEM((4, UP), jnp.float32),
          pltpu.VMEM((UP,), jnp.float32),
      ],
  )
  return k(s, up, mp)


def _val_body(s_hbm, u_hbm, i_hbm, r_hbm, mp_hbm, ssp_hbm, val_hbm, p_hbm,
              mvec, svec, mtmp, sbuf, ubuf, ibuf, rbuf, vout, pout):
  cid = lax.axis_index("c")
  sid = lax.axis_index("s")
  wid = sid * NC + cid
  base = wid * EW

  _combine_partials(mp_hbm, mtmp, mvec, True, 4)
  _combine_partials(ssp_hbm, mtmp, svec, False, 4)

  pltpu.sync_copy(s_hbm.at[pl.ds(base, EW)], sbuf)
  pltpu.sync_copy(u_hbm.at[pl.ds(base, EW)], ubuf)
  pltpu.sync_copy(i_hbm.at[pl.ds(base, EW)], ibuf)
  pltpu.sync_copy(r_hbm.at[pl.ds(base, EW)], rbuf)

  def grp(g, c):
    o = g * 16
    uv = ubuf[pl.ds(o, 16)]
    sv = sbuf[pl.ds(o, 16)]
    iv = ibuf[pl.ds(o, 16)]
    rv = rbuf[pl.ds(o, 16)]
    mv = plsc.load_gather(mvec, [uv])
    ssv = plsc.load_gather(svec, [uv])
    vout[pl.ds(o, 16)] = jnp.exp(sv - mv) / (ssv + 1e-9)
    pout[pl.ds(o, 16)] = rv * ITEMS + iv
    return c
  lax.fori_loop(0, EW // 16, grp, 0)
  pltpu.sync_copy(vout, val_hbm.at[pl.ds(base, EW)])
  pltpu.sync_copy(pout, p_hbm.at[pl.ds(base, EW)])


def _run_val(s, up, ip, rp, mp, ssp):
  k = pl.kernel(
      _val_body,
      out_type=(jax.ShapeDtypeStruct((EP,), jnp.float32),
                jax.ShapeDtypeStruct((EP,), jnp.int32)),
      mesh=_mesh(),
      compiler_params=_SC_PARAMS,
      scratch_types=[
          pltpu.VMEM((UP,), jnp.float32),
          pltpu.VMEM((UP,), jnp.float32),
          pltpu.VMEM((4, UP), jnp.float32),
          pltpu.VMEM((EW,), jnp.float32),
          pltpu.VMEM((EW,), jnp.int32),
          pltpu.VMEM((EW,), jnp.int32),
          pltpu.VMEM((EW,), jnp.int32),
          pltpu.VMEM((EW,), jnp.float32),
          pltpu.VMEM((EW,), jnp.int32),
      ],
  )
  return k(s, up, ip, rp, mp, ssp)


UO = UP // NW
ST = 4096
NST = EP // ST
C5 = 64
IOTA = lambda: lax.iota(jnp.int32, 16)


def _agg_body(u_hbm, val_hbm, p_hbm, xia_hbm, nf_hbm,
              uA, vA, pA, uB, vB, pB, mu, mv, mp_, rowsA, rowsB, acc,
              semA, semB, semRA, semRB):
  cid = lax.axis_index("c")
  sid = lax.axis_index("s")
  wid = sid * NC + cid
  lo = wid * UO
  hi = jnp.minimum(lo + UO, USERS)

  def zrow(j, c):
    for cc in range(D // 16):
      acc[j, pl.ds(cc * 16, 16)] = jnp.zeros((16,), jnp.float32)
    return c
  lax.fori_loop(0, UO, zrow, 0)

  def zmatch(t, c):
    mu[pl.ds(t * 16, 16)] = jnp.full((16,), lo, jnp.int32)
    mv[pl.ds(t * 16, 16)] = jnp.zeros((16,), jnp.float32)
    mp_[pl.ds(t * 16, 16)] = jnp.zeros((16,), jnp.int32)
    return c
  lax.fori_loop(0, (ST + 16) // 16, zmatch, 0)

  def fire_strip(st, ub, vb, pb, sem):
    base = st * ST
    pltpu.make_async_copy(u_hbm.at[pl.ds(base, ST)], ub, sem).start()
    pltpu.make_async_copy(val_hbm.at[pl.ds(base, ST)], vb, sem).start()
    pltpu.make_async_copy(p_hbm.at[pl.ds(base, ST)], pb, sem).start()

  def drain_strip(ub, vb, pb, sem):
    pltpu.make_async_copy(u_hbm.at[pl.ds(0, ST)], ub, sem).wait()
    pltpu.make_async_copy(val_hbm.at[pl.ds(0, ST)], vb, sem).wait()
    pltpu.make_async_copy(p_hbm.at[pl.ds(0, ST)], pb, sem).wait()

  def do_strip(ub, vb, pb):
    def scan(g, pos):
      o = g * 16
      uv = ub[pl.ds(o, 16)]
      mask = (uv >= lo) & (uv < hi)
      cnt = plsc.all_reduce_population_count(mask)[0]
      plsc.store_compressed(mu.at[pl.ds(pos, 16)], uv, mask=mask)
      plsc.store_compressed(mv.at[pl.ds(pos, 16)], vb[pl.ds(o, 16)],
                            mask=mask)
      plsc.store_compressed(mp_.at[pl.ds(pos, 16)], pb[pl.ds(o, 16)],
                            mask=mask)
      return pos + cnt
    pos = lax.fori_loop(0, ST // 16, scan, jnp.int32(0))

    nch = (pos + C5 - 1) // C5

    def fire_rows(ch, rbuf, sem):
      pltpu.make_async_copy(xia_hbm.at[mp_.at[pl.ds(ch * C5, C5)]], rbuf,
                            sem).start()

    def zero_tail(ch):
      for g in range(C5 // 16):
        o = ch * C5 + g * 16
        keep = (o + IOTA()) < pos
        mv[pl.ds(o, 16)] = jnp.where(keep, mv[pl.ds(o, 16)], 0.0)

    def do_chunk(ch, rbuf):
      k0 = ch * C5

      gmode = lax.GatherScatterMode.PROMISE_IN_BOUNDS

      def egrp(g, c3):
        o = g * 16
        lu_vec = mu[pl.ds(k0 + o, 16)] - lo
        val_vec = mv[pl.ds(k0 + o, 16)]
        for j in range(16):
          jc = jnp.full((16,), j, jnp.int32)
          row_sp = lu_vec.at[jc].get(mode=gmode)
          val_sp = val_vec.at[jc].get(mode=gmode)
          loads = [rbuf[g * 16 + j, pl.ds(cc * 16, 16)]
                   for cc in range(D // 16)]
          col0 = IOTA()
          for cc in range(D // 16):
            plsc.addupdate_scatter(acc, [row_sp, col0 + (cc * 16)],
                                   loads[cc] * val_sp)
        return c3
      lax.fori_loop(0, C5 // 16, egrp, 0)

    @pl.when(nch > 0)
    def _():
      zero_tail(nch - 1)
      fire_rows(0, rowsA, semRA)

      def pair(t, c):
        c0 = 2 * t
        c1 = 2 * t + 1

        @pl.when(c1 < nch)
        def _():
          fire_rows(c1, rowsB, semRB)
        pltpu.make_async_copy(xia_hbm.at[mp_.at[pl.ds(0, C5)]], rowsA,
                              semRA).wait()
        do_chunk(c0, rowsA)

        @pl.when(c0 + 2 < nch)
        def _():
          fire_rows(c0 + 2, rowsA, semRA)

        @pl.when(c1 < nch)
        def _():
          pltpu.make_async_copy(xia_hbm.at[mp_.at[pl.ds(0, C5)]], rowsB,
                                semRB).wait()
          do_chunk(c1, rowsB)
        return c
      lax.fori_loop(0, (nch + 1) // 2, pair, 0)

  fire_strip(0, uA, vA, pA, semA)

  def strips(t, c):
    st0 = 2 * t
    fire_strip(st0 + 1, uB, vB, pB, semB)
    drain_strip(uA, vA, pA, semA)
    do_strip(uA, vA, pA)

    @pl.when(st0 + 2 < NST)
    def _():
      fire_strip(st0 + 2, uA, vA, pA, semA)
    drain_strip(uB, vB, pB, semB)
    do_strip(uB, vB, pB)
    return c
  lax.fori_loop(0, NST // 2, strips, 0)

  pltpu.sync_copy(acc, nf_hbm.at[pl.ds(lo, UO)])


def _run_agg(up, val, p, xia_flat):
  k = pl.kernel(
      _agg_body,
      out_type=jax.ShapeDtypeStruct((UP, D), jnp.float32),
      mesh=_mesh(),
      compiler_params=_SC_PARAMS,
      scratch_types=[
          pltpu.VMEM((ST,), jnp.int32),
          pltpu.VMEM((ST,), jnp.float32),
          pltpu.VMEM((ST,), jnp.int32),
          pltpu.VMEM((ST,), jnp.int32),
          pltpu.VMEM((ST,), jnp.float32),
          pltpu.VMEM((ST,), jnp.int32),
          pltpu.VMEM((ST + 16,), jnp.int32),
          pltpu.VMEM((ST + 16,), jnp.float32),
          pltpu.VMEM((ST + 16,), jnp.int32),
          pltpu.VMEM((C5, D), jnp.float32),
          pltpu.VMEM((C5, D), jnp.float32),
          pltpu.VMEM((UO, D), jnp.float32),
          pltpu.SemaphoreType.DMA,
          pltpu.SemaphoreType.DMA,
          pltpu.SemaphoreType.DMA,
          pltpu.SemaphoreType.DMA,
      ],
  )
  return k(up, val, p, xia_flat)


def _out_body(nf_ref, w_ref, b_ref, out_ref):
  out_ref[...] = jnp.dot(nf_ref[...], w_ref[...],
                         preferred_element_type=jnp.float32) + b_ref[...]


def _run_out(nf, wT, wb):
  return pl.pallas_call(
      _out_body,
      grid=(5,),
      in_specs=[
          pl.BlockSpec((1000, D), lambda b: (b, 0)),
          pl.BlockSpec((D, D), lambda b: (0, 0)),
          pl.BlockSpec((1, D), lambda b: (0, 0)),
      ],
      out_specs=pl.BlockSpec((1000, D), lambda b: (b, 0)),
      out_shape=jax.ShapeDtypeStruct((USERS, D), jnp.float32),
  )(nf, wT, wb)


@jax.jit
def kernel(i, u, rating, user_feat, item_feat, rating_feat,
           gv_w1, gv_b1, gv_w2, gv_b2,
           att1_w, att1_b, att2_w, att2_b, att3_w, att3_b,
           w_w, w_b):
  i = i.astype(jnp.int32)
  u = u.astype(jnp.int32)
  rating = rating.astype(jnp.int32)

  pad = EP - E
  ip = jnp.concatenate([i, jnp.zeros((pad,), jnp.int32)])
  rp = jnp.concatenate([rating, jnp.zeros((pad,), jnp.int32)])
  up = jnp.concatenate([u, jnp.full((pad,), UP - 1, jnp.int32)])

  rating_pad = jnp.concatenate(
      [rating_feat, jnp.zeros((8 - NR, D), jnp.float32)], axis=0)
  w1iT = gv_w1[:, :D].T
  w1rT = gv_w1[:, D:].T
  b1 = gv_b1.reshape(1, D)
  w2T = gv_w2.T
  b2 = gv_b2.reshape(1, D)
  a1xT = att1_w[:, :D].T
  a1uT = att1_w[:, D:].T
  a1b = att1_b.reshape(1, D)
  a2T = att2_w.T
  a2b = att2_b.reshape(1, D)
  a3 = att3_w.reshape(1, D)
  wT = w_w.T
  wb = w_b.reshape(1, D)

  xia_tab, xa1_tab, up1 = _build_tables(
      item_feat, user_feat, rating_pad, w1iT, w1rT, b1, w2T, b2, a1xT, a1uT,
      a1b)
  xia_flat = xia_tab.reshape(NR * ITEMS, D)
  xa1_flat = xa1_tab.reshape(NR * ITEMS, D)

  g = _run_gather(xa1_flat, up1, ip, rp, up)
  s = _run_score(g, a2T, a2b, a3)
  mp = _run_segmax(s, up)
  ssp = _run_segsum(s, up, mp)
  val, p = _run_val(s, up, ip, rp, mp, ssp)
  nf = _run_agg(up, val, p, xia_flat)
  return _run_out(nf, wT, wb)


# speedup vs baseline: 1.6989x; 1.6989x over previous
import jax
import jax.numpy as jnp
from jax import lax
from jax.experimental import pallas as pl
from jax.experimental.pallas import tpu as pltpu
from jax.experimental.pallas import tpu_sc as plsc

E = 160000
USERS = 5000
ITEMS = 5000
D = 256
NR = 5

NC = 2
NS = 16
NW = NC * NS
EP = 163840
EW = EP // NW
CH = 128
NCH = EW // CH
UP = 5120
USC = UP // NS
C5 = 64

_mesh = lambda: plsc.VectorSubcoreMesh(
    core_axis_name="c", subcore_axis_name="s", num_cores=NC, num_subcores=NS)
_SC_PARAMS = pltpu.CompilerParams(needs_layout_passes=False)


def _leaky(x):
  return jnp.where(x >= 0, x, 0.01 * x)


def _tables_body(item_ref, user_ref, rating_ref, w1i_ref, w1r_ref, b1_ref,
                 w2_ref, b2_ref, a1x_ref, a1u_ref, a1b_ref,
                 xia_ref, xa1_ref, up1_ref):
  r = pl.program_id(0)
  rp1_all = jnp.dot(rating_ref[...], w1r_ref[...],
                    preferred_element_type=jnp.float32) + b1_ref[...]
  onehot = (lax.broadcasted_iota(jnp.int32, (1, 8), 1) == r).astype(jnp.float32)
  rp1 = jnp.dot(onehot, rp1_all, preferred_element_type=jnp.float32)
  ip1 = jnp.dot(item_ref[...], w1i_ref[...], preferred_element_type=jnp.float32)
  h1 = _leaky(ip1 + rp1)
  xia = _leaky(jnp.dot(h1, w2_ref[...], preferred_element_type=jnp.float32)
               + b2_ref[...])
  xia_ref[...] = xia[None]
  xa1_ref[...] = jnp.dot(xia, a1x_ref[...],
                         preferred_element_type=jnp.float32)[None]
  up1_ref[...] = jnp.dot(user_ref[...], a1u_ref[...],
                         preferred_element_type=jnp.float32) + a1b_ref[...]


def _build_tables(item_feat, user_feat, rating_pad, w1iT, w1rT, b1, w2T, b2,
                  a1xT, a1uT, a1b):
  nb = 5
  full = lambda *dims: pl.BlockSpec(dims, lambda r, b: tuple(0 for _ in dims))
  return pl.pallas_call(
      _tables_body,
      grid=(NR, nb),
      in_specs=[
          pl.BlockSpec((1000, D), lambda r, b: (b, 0)),
          pl.BlockSpec((1000, D), lambda r, b: (b, 0)),
          full(8, D), full(D, D), full(D, D), full(1, D),
          full(D, D), full(1, D), full(D, D), full(D, D), full(1, D),
      ],
      out_specs=[
          pl.BlockSpec((1, 1000, D), lambda r, b: (r, b, 0)),
          pl.BlockSpec((1, 1000, D), lambda r, b: (r, b, 0)),
          pl.BlockSpec((1000, D), lambda r, b: (b, 0)),
      ],
      out_shape=[
          jax.ShapeDtypeStruct((NR, ITEMS, D), jnp.float32),
          jax.ShapeDtypeStruct((NR, ITEMS, D), jnp.float32),
          jax.ShapeDtypeStruct((USERS, D), jnp.float32),
      ],
  )(item_feat, user_feat, rating_pad, w1iT, w1rT, b1, w2T, b2, a1xT, a1uT, a1b)


def _gather_body(xa1_hbm, up1_hbm, i_hbm, r_hbm, u_hbm, g_hbm,
                 ibuf, rbuf, ubuf, pbuf, rowsA, rowsB, semA, semB):
  cid = lax.axis_index("c")
  sid = lax.axis_index("s")
  wid = sid * NC + cid
  base = wid * EW
  pltpu.sync_copy(i_hbm.at[pl.ds(base, EW)], ibuf)
  pltpu.sync_copy(r_hbm.at[pl.ds(base, EW)], rbuf)
  pltpu.sync_copy(u_hbm.at[pl.ds(base, EW)], ubuf)

  def chunk(c, carry):
    off = c * CH
    for g in range(CH // 16):
      o = off + g * 16
      iv = ibuf[pl.ds(o, 16)]
      rv = rbuf[pl.ds(o, 16)]
      pbuf[pl.ds(g * 16, 16)] = rv * ITEMS + iv
    cpA = pltpu.async_copy(xa1_hbm.at[pbuf], rowsA, semA)
    cpB = pltpu.async_copy(up1_hbm.at[ubuf.at[pl.ds(off, CH)]], rowsB, semB)
    cpA.wait()
    cpB.wait()

    def row(j, rc):
      for cc in range(D // 16):
        x = rowsA[j, pl.ds(cc * 16, 16)] + rowsB[j, pl.ds(cc * 16, 16)]
        rowsA[j, pl.ds(cc * 16, 16)] = jnp.maximum(x, 0.0)
      return rc
    lax.fori_loop(0, CH, row, 0)
    pltpu.sync_copy(rowsA, g_hbm.at[pl.ds(base + off, CH)])
    return carry
  lax.fori_loop(0, NCH, chunk, 0)


def _run_gather(xa1_flat, up1, ip, rp, up):
  k = pl.kernel(
      _gather_body,
      out_type=jax.ShapeDtypeStruct((EP, D), jnp.float32),
      mesh=_mesh(),
      compiler_params=_SC_PARAMS,
      scratch_types=[
          pltpu.VMEM((EW,), jnp.int32),
          pltpu.VMEM((EW,), jnp.int32),
          pltpu.VMEM((EW,), jnp.int32),
          pltpu.VMEM((CH,), jnp.int32),
          pltpu.VMEM((CH, D), jnp.float32),
          pltpu.VMEM((CH, D), jnp.float32),
          pltpu.SemaphoreType.DMA,
          pltpu.SemaphoreType.DMA,
      ],
  )
  return k(xa1_flat, up1, ip, rp, up)


def _score_body(g_ref, a2_ref, a2b_ref, a3_ref, out_ref):
  a2 = jnp.maximum(
      jnp.dot(g_ref[...], a2_ref[...], preferred_element_type=jnp.float32)
      + a2b_ref[...], 0.0)
  s = jnp.sum(a2 * a3_ref[...], axis=1)
  out_ref[...] = s.reshape(1, 1, 1280)


def _run_score(g, a2T, a2b, a3):
  nb = EP // 1280
  s3 = pl.pallas_call(
      _score_body,
      grid=(nb,),
      in_specs=[
          pl.BlockSpec((1280, D), lambda b: (b, 0)),
          pl.BlockSpec((D, D), lambda b: (0, 0)),
          pl.BlockSpec((1, D), lambda b: (0, 0)),
          pl.BlockSpec((1, D), lambda b: (0, 0)),
      ],
      out_specs=pl.BlockSpec((1, 1, 1280), lambda b: (b, 0, 0)),
      out_shape=jax.ShapeDtypeStruct((nb, 1, 1280), jnp.float32),
  )(g, a2T, a2b, a3)
  return s3.reshape(EP)


def _segmax_body(s_hbm, u_hbm, mp_hbm, acc, sbuf, ubuf, mrow):
  cid = lax.axis_index("c")
  sid = lax.axis_index("s")
  wid = sid * NC + cid
  base = wid * EW

  def init(k, c):
    acc[pl.ds(k * 16, 16)] = jnp.full((16,), -1e30, jnp.float32)
    return c
  lax.fori_loop(0, (16 * UP) // 16, init, 0)

  pltpu.sync_copy(s_hbm.at[pl.ds(base, EW)], sbuf)
  pltpu.sync_copy(u_hbm.at[pl.ds(base, EW)], ubuf)
  lanes = lax.iota(jnp.int32, 16) * UP

  def grp(g, c):
    uv = ubuf[pl.ds(g * 16, 16)]
    sv = sbuf[pl.ds(g * 16, 16)]
    idx = lanes + uv
    cur = plsc.load_gather(acc, [idx])
    plsc.store_scatter(acc, [idx], jnp.maximum(cur, sv))
    return c
  lax.fori_loop(0, EW // 16, grp, 0)

  def red(t, c):
    b = t * 16
    m = acc[pl.ds(b, 16)]
    for l in range(1, 16):
      m = jnp.maximum(m, acc[pl.ds(l * UP + b, 16)])
    mrow[pl.ds(b, 16)] = m
    return c
  lax.fori_loop(0, UP // 16, red, 0)
  pltpu.sync_copy(mrow, mp_hbm.at[wid])


def _run_segmax(s, up):
  k = pl.kernel(
      _segmax_body,
      out_type=jax.ShapeDtypeStruct((NW, UP), jnp.float32),
      mesh=_mesh(),
      compiler_params=_SC_PARAMS,
      scratch_types=[
          pltpu.VMEM((16 * UP,), jnp.float32),
          pltpu.VMEM((EW,), jnp.float32),
          pltpu.VMEM((EW,), jnp.int32),
          pltpu.VMEM((UP,), jnp.float32),
      ],
  )
  return k(s, up)


def _combine_partials(src_hbm, mtmp, dst, is_max, rows_per_dma):
  neutral = -1e30 if is_max else 0.0

  def init(t, c):
    dst[pl.ds(t * 16, 16)] = jnp.full((16,), neutral, jnp.float32)
    return c
  lax.fori_loop(0, UP // 16, init, 0)
  for blk in range(NW // rows_per_dma):
    pltpu.sync_copy(src_hbm.at[pl.ds(blk * rows_per_dma, rows_per_dma)], mtmp)

    def red(t, c):
      b = t * 16
      v = dst[pl.ds(b, 16)]
      for r in range(rows_per_dma):
        w = mtmp[r, pl.ds(b, 16)]
        v = jnp.maximum(v, w) if is_max else v + w
      dst[pl.ds(b, 16)] = v
      return c
    lax.fori_loop(0, UP // 16, red, 0)


def _segsum_body(s_hbm, u_hbm, mp_hbm, ssp_hbm, acc, sbuf, ubuf, mvec, mtmp,
                 srow):
  cid = lax.axis_index("c")
  sid = lax.axis_index("s")
  wid = sid * NC + cid
  base = wid * EW

  _combine_partials(mp_hbm, mtmp, mvec, True, 4)

  def init(k, c):
    acc[pl.ds(k * 16, 16)] = jnp.zeros((16,), jnp.float32)
    return c
  lax.fori_loop(0, (16 * UP) // 16, init, 0)

  pltpu.sync_copy(s_hbm.at[pl.ds(base, EW)], sbuf)
  pltpu.sync_copy(u_hbm.at[pl.ds(base, EW)], ubuf)
  lanes = lax.iota(jnp.int32, 16) * UP

  def grp(g, c):
    uv = ubuf[pl.ds(g * 16, 16)]
    sv = sbuf[pl.ds(g * 16, 16)]
    mv = plsc.load_gather(mvec, [uv])
    ex = jnp.exp(sv - mv)
    plsc.addupdate_scatter(acc, [lanes + uv], ex)
    return c
  lax.fori_loop(0, EW // 16, grp, 0)

  def red(t, c):
    b = t * 16
    v = acc[pl.ds(b, 16)]
    for l in range(1, 16):
      v = v + acc[pl.ds(l * UP + b, 16)]
    srow[pl.ds(b, 16)] = v
    return c
  lax.fori_loop(0, UP // 16, red, 0)
  pltpu.sync_copy(srow, ssp_hbm.at[wid])


def _run_segsum(s, up, mp):
  k = pl.kernel(
      _segsum_body,
      out_type=jax.ShapeDtypeStruct((NW, UP), jnp.float32),
      mesh=_mesh(),
      compiler_params=_SC_PARAMS,
      scratch_types=[
          pltpu.VMEM((16 * UP,), jnp.float32),
          pltpu.VMEM((EW,), jnp.float32),
          pltpu.VMEM((EW,), jnp.int32),
          pltpu.VMEM((UP,), jnp.float32),
          pltpu.VMEM((4, UP), jnp.float32),
          pltpu.VMEM((UP,), jnp.float32),
      ],
  )
  return k(s, up, mp)


def _val_body(s_hbm, u_hbm, i_hbm, r_hbm, mp_hbm, ssp_hbm, val_hbm, p_hbm,
              mvec, svec, mtmp, sbuf, ubuf, ibuf, rbuf, vout, pout):
  cid = lax.axis_index("c")
  sid = lax.axis_index("s")
  wid = sid * NC + cid
  base = wid * EW

  _combine_partials(mp_hbm, mtmp, mvec, True, 4)
  _combine_partials(ssp_hbm, mtmp, svec, False, 4)

  pltpu.sync_copy(s_hbm.at[pl.ds(base, EW)], sbuf)
  pltpu.sync_copy(u_hbm.at[pl.ds(base, EW)], ubuf)
  pltpu.sync_copy(i_hbm.at[pl.ds(base, EW)], ibuf)
  pltpu.sync_copy(r_hbm.at[pl.ds(base, EW)], rbuf)

  def grp(g, c):
    o = g * 16
    uv = ubuf[pl.ds(o, 16)]
    sv = sbuf[pl.ds(o, 16)]
    iv = ibuf[pl.ds(o, 16)]
    rv = rbuf[pl.ds(o, 16)]
    mv = plsc.load_gather(mvec, [uv])
    ssv = plsc.load_gather(svec, [uv])
    vout[pl.ds(o, 16)] = jnp.exp(sv - mv) / (ssv + 1e-9)
    pout[pl.ds(o, 16)] = rv * ITEMS + iv
    return c
  lax.fori_loop(0, EW // 16, grp, 0)
  pltpu.sync_copy(vout, val_hbm.at[pl.ds(base, EW)])
  pltpu.sync_copy(pout, p_hbm.at[pl.ds(base, EW)])


def _run_val(s, up, ip, rp, mp, ssp):
  k = pl.kernel(
      _val_body,
      out_type=(jax.ShapeDtypeStruct((EP,), jnp.float32),
                jax.ShapeDtypeStruct((EP,), jnp.int32)),
      mesh=_mesh(),
      compiler_params=_SC_PARAMS,
      scratch_types=[
          pltpu.VMEM((UP,), jnp.float32),
          pltpu.VMEM((UP,), jnp.float32),
          pltpu.VMEM((4, UP), jnp.float32),
          pltpu.VMEM((EW,), jnp.float32),
          pltpu.VMEM((EW,), jnp.int32),
          pltpu.VMEM((EW,), jnp.int32),
          pltpu.VMEM((EW,), jnp.int32),
          pltpu.VMEM((EW,), jnp.float32),
          pltpu.VMEM((EW,), jnp.int32),
      ],
  )
  return k(s, up, ip, rp, mp, ssp)


UO = UP // NW
ST = 4096
NST = EP // ST
FM = 5120
FMB = 5248
C5 = 96
IOTA = lambda: lax.iota(jnp.int32, 16)


def _agg_body(u_hbm, val_hbm, p_hbm, xia_hbm, nf_hbm,
              uA, vA, pA, uB, vB, pB, mu, mv, mp_, rowsA, rowsB, acc,
              semA, semB, semRA, semRB):
  cid = lax.axis_index("c")
  sid = lax.axis_index("s")
  wid = sid * NC + cid
  lo = wid * UO
  hi = jnp.minimum(lo + UO, USERS)

  def zrow(j, c):
    for cc in range(D // 16):
      acc[j, pl.ds(cc * 16, 16)] = jnp.zeros((16,), jnp.float32)
    return c
  lax.fori_loop(0, UO, zrow, 0)

  def zmatch(t, c):
    mu[pl.ds(t * 16, 16)] = jnp.full((16,), lo, jnp.int32)
    mv[pl.ds(t * 16, 16)] = jnp.zeros((16,), jnp.float32)
    mp_[pl.ds(t * 16, 16)] = jnp.zeros((16,), jnp.int32)
    return c
  lax.fori_loop(0, FMB // 16, zmatch, 0)

  def fire_strip(st, ub, vb, pb, sem):
    base = st * ST
    pltpu.make_async_copy(u_hbm.at[pl.ds(base, ST)], ub, sem).start()
    pltpu.make_async_copy(val_hbm.at[pl.ds(base, ST)], vb, sem).start()
    pltpu.make_async_copy(p_hbm.at[pl.ds(base, ST)], pb, sem).start()

  def drain_strip(ub, vb, pb, sem):
    pltpu.make_async_copy(u_hbm.at[pl.ds(0, ST)], ub, sem).wait()
    pltpu.make_async_copy(val_hbm.at[pl.ds(0, ST)], vb, sem).wait()
    pltpu.make_async_copy(p_hbm.at[pl.ds(0, ST)], pb, sem).wait()

  def flush(pos):
    nch = (pos + C5 - 1) // C5

    def fire_rows(ch, rbuf, sem):
      pltpu.make_async_copy(xia_hbm.at[mp_.at[pl.ds(ch * C5, C5)]], rbuf,
                            sem).start()

    def zero_tail(ch):
      for g in range(C5 // 16):
        o = ch * C5 + g * 16
        keep = (o + IOTA()) < pos
        mv[pl.ds(o, 16)] = jnp.where(keep, mv[pl.ds(o, 16)], 0.0)

    def do_chunk(ch, rbuf):
      k0 = ch * C5

      gmode = lax.GatherScatterMode.PROMISE_IN_BOUNDS

      def egrp(g, c3):
        o = g * 16
        lu_vec = mu[pl.ds(k0 + o, 16)] - lo
        val_vec = mv[pl.ds(k0 + o, 16)]
        for j in range(16):
          jc = jnp.full((16,), j, jnp.int32)
          row_sp = lu_vec.at[jc].get(mode=gmode)
          val_sp = val_vec.at[jc].get(mode=gmode)
          loads = [rbuf[g * 16 + j, pl.ds(cc * 16, 16)]
                   for cc in range(D // 16)]
          col0 = IOTA()
          for cc in range(D // 16):
            plsc.addupdate_scatter(acc, [row_sp, col0 + (cc * 16)],
                                   loads[cc] * val_sp)
        return c3
      lax.fori_loop(0, C5 // 16, egrp, 0)

    @pl.when(nch > 0)
    def _():
      zero_tail(nch - 1)
      fire_rows(0, rowsA, semRA)

      def pair(t, c):
        c0 = 2 * t
        c1 = 2 * t + 1

        @pl.when(c1 < nch)
        def _():
          fire_rows(c1, rowsB, semRB)
        pltpu.make_async_copy(xia_hbm.at[mp_.at[pl.ds(0, C5)]], rowsA,
                              semRA).wait()
        do_chunk(c0, rowsA)

        @pl.when(c0 + 2 < nch)
        def _():
          fire_rows(c0 + 2, rowsA, semRA)

        @pl.when(c1 < nch)
        def _():
          pltpu.make_async_copy(xia_hbm.at[mp_.at[pl.ds(0, C5)]], rowsB,
                                semRB).wait()
          do_chunk(c1, rowsB)
        return c
      lax.fori_loop(0, (nch + 1) // 2, pair, 0)
    return jnp.int32(0)

  def do_strip(ub, vb, pb, pos0):
    def scan(g, pos):
      o = g * 16
      uv = ub[pl.ds(o, 16)]
      mask = (uv >= lo) & (uv < hi)
      cnt = plsc.all_reduce_population_count(mask)[0]
      plsc.store_compressed(mu.at[pl.ds(pos, 16)], uv, mask=mask)
      plsc.store_compressed(mv.at[pl.ds(pos, 16)], vb[pl.ds(o, 16)],
                            mask=mask)
      plsc.store_compressed(mp_.at[pl.ds(pos, 16)], pb[pl.ds(o, 16)],
                            mask=mask)
      return pos + cnt
    pos = lax.fori_loop(0, ST // 16, scan, pos0)
    return lax.cond(pos > FM - ST, flush, lambda p: p, pos)

  fire_strip(0, uA, vA, pA, semA)

  def strips(t, pos):
    st0 = 2 * t
    fire_strip(st0 + 1, uB, vB, pB, semB)
    drain_strip(uA, vA, pA, semA)
    pos = do_strip(uA, vA, pA, pos)

    @pl.when(st0 + 2 < NST)
    def _():
      fire_strip(st0 + 2, uA, vA, pA, semA)
    drain_strip(uB, vB, pB, semB)
    return do_strip(uB, vB, pB, pos)
  pos = lax.fori_loop(0, NST // 2, strips, jnp.int32(0))
  flush(pos)

  pltpu.sync_copy(acc, nf_hbm.at[pl.ds(lo, UO)])


def _run_agg(up, val, p, xia_flat):
  k = pl.kernel(
      _agg_body,
      out_type=jax.ShapeDtypeStruct((UP, D), jnp.float32),
      mesh=_mesh(),
      compiler_params=_SC_PARAMS,
      scratch_types=[
          pltpu.VMEM((ST,), jnp.int32),
          pltpu.VMEM((ST,), jnp.float32),
          pltpu.VMEM((ST,), jnp.int32),
          pltpu.VMEM((ST,), jnp.int32),
          pltpu.VMEM((ST,), jnp.float32),
          pltpu.VMEM((ST,), jnp.int32),
          pltpu.VMEM((FMB,), jnp.int32),
          pltpu.VMEM((FMB,), jnp.float32),
          pltpu.VMEM((FMB,), jnp.int32),
          pltpu.VMEM((C5, D), jnp.float32),
          pltpu.VMEM((C5, D), jnp.float32),
          pltpu.VMEM((UO, D), jnp.float32),
          pltpu.SemaphoreType.DMA,
          pltpu.SemaphoreType.DMA,
          pltpu.SemaphoreType.DMA,
          pltpu.SemaphoreType.DMA,
      ],
  )
  return k(up, val, p, xia_flat)


def _out_body(nf_ref, w_ref, b_ref, out_ref):
  out_ref[...] = jnp.dot(nf_ref[...], w_ref[...],
                         preferred_element_type=jnp.float32) + b_ref[...]


def _run_out(nf, wT, wb):
  return pl.pallas_call(
      _out_body,
      grid=(5,),
      in_specs=[
          pl.BlockSpec((1000, D), lambda b: (b, 0)),
          pl.BlockSpec((D, D), lambda b: (0, 0)),
          pl.BlockSpec((1, D), lambda b: (0, 0)),
      ],
      out_specs=pl.BlockSpec((1000, D), lambda b: (b, 0)),
      out_shape=jax.ShapeDtypeStruct((USERS, D), jnp.float32),
  )(nf, wT, wb)


@jax.jit
def kernel(i, u, rating, user_feat, item_feat, rating_feat,
           gv_w1, gv_b1, gv_w2, gv_b2,
           att1_w, att1_b, att2_w, att2_b, att3_w, att3_b,
           w_w, w_b):
  i = i.astype(jnp.int32)
  u = u.astype(jnp.int32)
  rating = rating.astype(jnp.int32)

  pad = EP - E
  ip = jnp.concatenate([i, jnp.zeros((pad,), jnp.int32)])
  rp = jnp.concatenate([rating, jnp.zeros((pad,), jnp.int32)])
  up = jnp.concatenate([u, jnp.full((pad,), UP - 1, jnp.int32)])

  rating_pad = jnp.concatenate(
      [rating_feat, jnp.zeros((8 - NR, D), jnp.float32)], axis=0)
  w1iT = gv_w1[:, :D].T
  w1rT = gv_w1[:, D:].T
  b1 = gv_b1.reshape(1, D)
  w2T = gv_w2.T
  b2 = gv_b2.reshape(1, D)
  a1xT = att1_w[:, :D].T
  a1uT = att1_w[:, D:].T
  a1b = att1_b.reshape(1, D)
  a2T = att2_w.T
  a2b = att2_b.reshape(1, D)
  a3 = att3_w.reshape(1, D)
  wT = w_w.T
  wb = w_b.reshape(1, D)

  xia_tab, xa1_tab, up1 = _build_tables(
      item_feat, user_feat, rating_pad, w1iT, w1rT, b1, w2T, b2, a1xT, a1uT,
      a1b)
  xia_flat = xia_tab.reshape(NR * ITEMS, D)
  xa1_flat = xa1_tab.reshape(NR * ITEMS, D)

  g = _run_gather(xa1_flat, up1, ip, rp, up)
  s = _run_score(g, a2T, a2b, a3)
  mp = _run_segmax(s, up)
  ssp = _run_segsum(s, up, mp)
  val, p = _run_val(s, up, ip, rp, mp, ssp)
  nf = _run_agg(up, val, p, xia_flat)
  return _run_out(nf, wT, wb)


# speedup vs baseline: 1.8387x; 1.0823x over previous
import jax
import jax.numpy as jnp
from jax import lax
from jax.experimental import pallas as pl
from jax.experimental.pallas import tpu as pltpu
from jax.experimental.pallas import tpu_sc as plsc

E = 160000
USERS = 5000
ITEMS = 5000
D = 256
NR = 5

NC = 2
NS = 16
NW = NC * NS
EP = 163840
EW = EP // NW
CH = 128
C2 = 64
UP = 5120
USC = UP // NS
C5 = 64

_mesh = lambda: plsc.VectorSubcoreMesh(
    core_axis_name="c", subcore_axis_name="s", num_cores=NC, num_subcores=NS)
_SC_PARAMS = pltpu.CompilerParams(needs_layout_passes=False)


def _leaky(x):
  return jnp.where(x >= 0, x, 0.01 * x)


def _tables_body(item_ref, user_ref, rating_ref, w1i_ref, w1r_ref, b1_ref,
                 w2_ref, b2_ref, a1x_ref, a1u_ref, a1b_ref,
                 xia_ref, xa1_ref, up1_ref):
  r = pl.program_id(0)
  rp1_all = jnp.dot(rating_ref[...], w1r_ref[...],
                    preferred_element_type=jnp.float32) + b1_ref[...]
  onehot = (lax.broadcasted_iota(jnp.int32, (1, 8), 1) == r).astype(jnp.float32)
  rp1 = jnp.dot(onehot, rp1_all, preferred_element_type=jnp.float32)
  ip1 = jnp.dot(item_ref[...], w1i_ref[...], preferred_element_type=jnp.float32)
  h1 = _leaky(ip1 + rp1)
  xia = _leaky(jnp.dot(h1, w2_ref[...], preferred_element_type=jnp.float32)
               + b2_ref[...])
  xia_ref[...] = xia[None]
  xa1_ref[...] = jnp.dot(xia, a1x_ref[...],
                         preferred_element_type=jnp.float32)[None]
  up1_ref[...] = jnp.dot(user_ref[...], a1u_ref[...],
                         preferred_element_type=jnp.float32) + a1b_ref[...]


def _build_tables(item_feat, user_feat, rating_pad, w1iT, w1rT, b1, w2T, b2,
                  a1xT, a1uT, a1b):
  nb = 5
  full = lambda *dims: pl.BlockSpec(dims, lambda r, b: tuple(0 for _ in dims))
  return pl.pallas_call(
      _tables_body,
      grid=(NR, nb),
      in_specs=[
          pl.BlockSpec((1000, D), lambda r, b: (b, 0)),
          pl.BlockSpec((1000, D), lambda r, b: (b, 0)),
          full(8, D), full(D, D), full(D, D), full(1, D),
          full(D, D), full(1, D), full(D, D), full(D, D), full(1, D),
      ],
      out_specs=[
          pl.BlockSpec((1, 1000, D), lambda r, b: (r, b, 0)),
          pl.BlockSpec((1, 1000, D), lambda r, b: (r, b, 0)),
          pl.BlockSpec((1000, D), lambda r, b: (b, 0)),
      ],
      out_shape=[
          jax.ShapeDtypeStruct((NR, ITEMS, D), jnp.float32),
          jax.ShapeDtypeStruct((NR, ITEMS, D), jnp.float32),
          jax.ShapeDtypeStruct((USERS, D), jnp.float32),
      ],
  )(item_feat, user_feat, rating_pad, w1iT, w1rT, b1, w2T, b2, a1xT, a1uT, a1b)


def _gather_body(xa1_hbm, up1_hbm, i_hbm, r_hbm, u_hbm, g_hbm,
                 ibuf, rbuf, ubuf, pbufA, pbufB, rowsAA, rowsBA, rowsAB,
                 rowsBB, outA, outB, semGA, semGB, semWA, semWB):
  cid = lax.axis_index("c")
  sid = lax.axis_index("s")
  wid = sid * NC + cid
  base = wid * EW
  pltpu.sync_copy(i_hbm.at[pl.ds(base, EW)], ibuf)
  pltpu.sync_copy(r_hbm.at[pl.ds(base, EW)], rbuf)
  pltpu.sync_copy(u_hbm.at[pl.ds(base, EW)], ubuf)

  def make_pbuf(c, pbuf):
    off = c * C2
    for g in range(C2 // 16):
      o = off + g * 16
      iv = ibuf[pl.ds(o, 16)]
      rv = rbuf[pl.ds(o, 16)]
      pbuf[pl.ds(g * 16, 16)] = rv * ITEMS + iv

  def fire_g(c, pbuf, ra, rb, sem):
    pltpu.make_async_copy(xa1_hbm.at[pbuf], ra, sem).start()
    pltpu.make_async_copy(up1_hbm.at[ubuf.at[pl.ds(c * C2, C2)]], rb,
                          sem).start()

  def wait_g(pbuf, ra, rb, sem):
    pltpu.make_async_copy(xa1_hbm.at[pbuf], ra, sem).wait()
    pltpu.make_async_copy(up1_hbm.at[pbuf], rb, sem).wait()

  def compute(ra, rb, ob):
    def row(j, rc):
      for cc in range(D // 16):
        x = ra[j, pl.ds(cc * 16, 16)] + rb[j, pl.ds(cc * 16, 16)]
        ob[j, pl.ds(cc * 16, 16)] = jnp.maximum(x, 0.0)
      return rc
    lax.fori_loop(0, C2, row, 0)

  def fire_w(c, ob, sem):
    pltpu.make_async_copy(ob, g_hbm.at[pl.ds(base + c * C2, C2)], sem).start()

  def drain_w(ob, sem):
    pltpu.make_async_copy(ob, g_hbm.at[pl.ds(base, C2)], sem).wait()

  make_pbuf(0, pbufA)
  fire_g(0, pbufA, rowsAA, rowsBA, semGA)
  nt = (EW // C2) // 2

  def body(t, carry):
    c0 = 2 * t
    c1 = 2 * t + 1

    @pl.when(t > 0)
    def _():
      drain_w(outB, semWB)
    make_pbuf(c1, pbufB)
    fire_g(c1, pbufB, rowsAB, rowsBB, semGB)
    wait_g(pbufA, rowsAA, rowsBA, semGA)
    compute(rowsAA, rowsBA, outA)
    fire_w(c0, outA, semWA)

    @pl.when(t + 1 < nt)
    def _():
      drain_w(outA, semWA)
      make_pbuf(c0 + 2, pbufA)
      fire_g(c0 + 2, pbufA, rowsAA, rowsBA, semGA)
    wait_g(pbufB, rowsAB, rowsBB, semGB)
    compute(rowsAB, rowsBB, outB)
    fire_w(c1, outB, semWB)
    return carry
  lax.fori_loop(0, nt, body, 0)
  drain_w(outA, semWA)
  drain_w(outB, semWB)


def _run_gather(xa1_flat, up1, ip, rp, up):
  k = pl.kernel(
      _gather_body,
      out_type=jax.ShapeDtypeStruct((EP, D), jnp.float32),
      mesh=_mesh(),
      compiler_params=_SC_PARAMS,
      scratch_types=[
          pltpu.VMEM((EW,), jnp.int32),
          pltpu.VMEM((EW,), jnp.int32),
          pltpu.VMEM((EW,), jnp.int32),
          pltpu.VMEM((C2,), jnp.int32),
          pltpu.VMEM((C2,), jnp.int32),
          pltpu.VMEM((C2, D), jnp.float32),
          pltpu.VMEM((C2, D), jnp.float32),
          pltpu.VMEM((C2, D), jnp.float32),
          pltpu.VMEM((C2, D), jnp.float32),
          pltpu.VMEM((C2, D), jnp.float32),
          pltpu.VMEM((C2, D), jnp.float32),
          pltpu.SemaphoreType.DMA,
          pltpu.SemaphoreType.DMA,
          pltpu.SemaphoreType.DMA,
          pltpu.SemaphoreType.DMA,
      ],
  )
  return k(xa1_flat, up1, ip, rp, up)


def _score_body(g_ref, a2_ref, a2b_ref, a3_ref, out_ref):
  a2 = jnp.maximum(
      jnp.dot(g_ref[...], a2_ref[...], preferred_element_type=jnp.float32)
      + a2b_ref[...], 0.0)
  s = jnp.sum(a2 * a3_ref[...], axis=1)
  out_ref[...] = s.reshape(1, 1, 1280)


def _run_score(g, a2T, a2b, a3):
  nb = EP // 1280
  s3 = pl.pallas_call(
      _score_body,
      grid=(nb,),
      in_specs=[
          pl.BlockSpec((1280, D), lambda b: (b, 0)),
          pl.BlockSpec((D, D), lambda b: (0, 0)),
          pl.BlockSpec((1, D), lambda b: (0, 0)),
          pl.BlockSpec((1, D), lambda b: (0, 0)),
      ],
      out_specs=pl.BlockSpec((1, 1, 1280), lambda b: (b, 0, 0)),
      out_shape=jax.ShapeDtypeStruct((nb, 1, 1280), jnp.float32),
  )(g, a2T, a2b, a3)
  return s3.reshape(EP)


def _segmax_body(s_hbm, u_hbm, mp_hbm, acc, sbuf, ubuf, mrow):
  cid = lax.axis_index("c")
  sid = lax.axis_index("s")
  wid = sid * NC + cid
  base = wid * EW

  def init(k, c):
    acc[pl.ds(k * 16, 16)] = jnp.full((16,), -1e30, jnp.float32)
    return c
  lax.fori_loop(0, (16 * UP) // 16, init, 0)

  pltpu.sync_copy(s_hbm.at[pl.ds(base, EW)], sbuf)
  pltpu.sync_copy(u_hbm.at[pl.ds(base, EW)], ubuf)
  lanes = lax.iota(jnp.int32, 16) * UP

  def grp(g, c):
    uv = ubuf[pl.ds(g * 16, 16)]
    sv = sbuf[pl.ds(g * 16, 16)]
    idx = lanes + uv
    cur = plsc.load_gather(acc, [idx])
    plsc.store_scatter(acc, [idx], jnp.maximum(cur, sv))
    return c
  lax.fori_loop(0, EW // 16, grp, 0)

  def red(t, c):
    b = t * 16
    m = acc[pl.ds(b, 16)]
    for l in range(1, 16):
      m = jnp.maximum(m, acc[pl.ds(l * UP + b, 16)])
    mrow[pl.ds(b, 16)] = m
    return c
  lax.fori_loop(0, UP // 16, red, 0)
  pltpu.sync_copy(mrow, mp_hbm.at[wid])


def _run_segmax(s, up):
  k = pl.kernel(
      _segmax_body,
      out_type=jax.ShapeDtypeStruct((NW, UP), jnp.float32),
      mesh=_mesh(),
      compiler_params=_SC_PARAMS,
      scratch_types=[
          pltpu.VMEM((16 * UP,), jnp.float32),
          pltpu.VMEM((EW,), jnp.float32),
          pltpu.VMEM((EW,), jnp.int32),
          pltpu.VMEM((UP,), jnp.float32),
      ],
  )
  return k(s, up)


def _combine_partials(src_hbm, mtmp, dst, is_max, rows_per_dma):
  neutral = -1e30 if is_max else 0.0

  def init(t, c):
    dst[pl.ds(t * 16, 16)] = jnp.full((16,), neutral, jnp.float32)
    return c
  lax.fori_loop(0, UP // 16, init, 0)
  for blk in range(NW // rows_per_dma):
    pltpu.sync_copy(src_hbm.at[pl.ds(blk * rows_per_dma, rows_per_dma)], mtmp)

    def red(t, c):
      b = t * 16
      v = dst[pl.ds(b, 16)]
      for r in range(rows_per_dma):
        w = mtmp[r, pl.ds(b, 16)]
        v = jnp.maximum(v, w) if is_max else v + w
      dst[pl.ds(b, 16)] = v
      return c
    lax.fori_loop(0, UP // 16, red, 0)


def _segsum_body(s_hbm, u_hbm, mp_hbm, ssp_hbm, acc, sbuf, ubuf, mvec, mtmp,
                 srow):
  cid = lax.axis_index("c")
  sid = lax.axis_index("s")
  wid = sid * NC + cid
  base = wid * EW

  _combine_partials(mp_hbm, mtmp, mvec, True, 4)

  def init(k, c):
    acc[pl.ds(k * 16, 16)] = jnp.zeros((16,), jnp.float32)
    return c
  lax.fori_loop(0, (16 * UP) // 16, init, 0)

  pltpu.sync_copy(s_hbm.at[pl.ds(base, EW)], sbuf)
  pltpu.sync_copy(u_hbm.at[pl.ds(base, EW)], ubuf)
  lanes = lax.iota(jnp.int32, 16) * UP

  def grp(g, c):
    uv = ubuf[pl.ds(g * 16, 16)]
    sv = sbuf[pl.ds(g * 16, 16)]
    mv = plsc.load_gather(mvec, [uv])
    ex = jnp.exp(sv - mv)
    plsc.addupdate_scatter(acc, [lanes + uv], ex)
    return c
  lax.fori_loop(0, EW // 16, grp, 0)

  def red(t, c):
    b = t * 16
    v = acc[pl.ds(b, 16)]
    for l in range(1, 16):
      v = v + acc[pl.ds(l * UP + b, 16)]
    srow[pl.ds(b, 16)] = v
    return c
  lax.fori_loop(0, UP // 16, red, 0)
  pltpu.sync_copy(srow, ssp_hbm.at[wid])


def _run_segsum(s, up, mp):
  k = pl.kernel(
      _segsum_body,
      out_type=jax.ShapeDtypeStruct((NW, UP), jnp.float32),
      mesh=_mesh(),
      compiler_params=_SC_PARAMS,
      scratch_types=[
          pltpu.VMEM((16 * UP,), jnp.float32),
          pltpu.VMEM((EW,), jnp.float32),
          pltpu.VMEM((EW,), jnp.int32),
          pltpu.VMEM((UP,), jnp.float32),
          pltpu.VMEM((4, UP), jnp.float32),
          pltpu.VMEM((UP,), jnp.float32),
      ],
  )
  return k(s, up, mp)


def _val_body(s_hbm, u_hbm, i_hbm, r_hbm, mp_hbm, ssp_hbm, val_hbm, p_hbm,
              mvec, svec, mtmp, sbuf, ubuf, ibuf, rbuf, vout, pout):
  cid = lax.axis_index("c")
  sid = lax.axis_index("s")
  wid = sid * NC + cid
  base = wid * EW

  _combine_partials(mp_hbm, mtmp, mvec, True, 4)
  _combine_partials(ssp_hbm, mtmp, svec, False, 4)

  pltpu.sync_copy(s_hbm.at[pl.ds(base, EW)], sbuf)
  pltpu.sync_copy(u_hbm.at[pl.ds(base, EW)], ubuf)
  pltpu.sync_copy(i_hbm.at[pl.ds(base, EW)], ibuf)
  pltpu.sync_copy(r_hbm.at[pl.ds(base, EW)], rbuf)

  def grp(g, c):
    o = g * 16
    uv = ubuf[pl.ds(o, 16)]
    sv = sbuf[pl.ds(o, 16)]
    iv = ibuf[pl.ds(o, 16)]
    rv = rbuf[pl.ds(o, 16)]
    mv = plsc.load_gather(mvec, [uv])
    ssv = plsc.load_gather(svec, [uv])
    vout[pl.ds(o, 16)] = jnp.exp(sv - mv) / (ssv + 1e-9)
    pout[pl.ds(o, 16)] = rv * ITEMS + iv
    return c
  lax.fori_loop(0, EW // 16, grp, 0)
  pltpu.sync_copy(vout, val_hbm.at[pl.ds(base, EW)])
  pltpu.sync_copy(pout, p_hbm.at[pl.ds(base, EW)])


def _run_val(s, up, ip, rp, mp, ssp):
  k = pl.kernel(
      _val_body,
      out_type=(jax.ShapeDtypeStruct((EP,), jnp.float32),
                jax.ShapeDtypeStruct((EP,), jnp.int32)),
      mesh=_mesh(),
      compiler_params=_SC_PARAMS,
      scratch_types=[
          pltpu.VMEM((UP,), jnp.float32),
          pltpu.VMEM((UP,), jnp.float32),
          pltpu.VMEM((4, UP), jnp.float32),
          pltpu.VMEM((EW,), jnp.float32),
          pltpu.VMEM((EW,), jnp.int32),
          pltpu.VMEM((EW,), jnp.int32),
          pltpu.VMEM((EW,), jnp.int32),
          pltpu.VMEM((EW,), jnp.float32),
          pltpu.VMEM((EW,), jnp.int32),
      ],
  )
  return k(s, up, ip, rp, mp, ssp)


UO = UP // NW
ST = 4096
NST = EP // ST
FM = 5120
FMB = 5248
C5 = 96
IOTA = lambda: lax.iota(jnp.int32, 16)


def _agg_body(u_hbm, val_hbm, p_hbm, xia_hbm, nf_hbm,
              uA, vA, pA, uB, vB, pB, mu, mv, mp_, rowsA, rowsB, acc,
              semA, semB, semRA, semRB):
  cid = lax.axis_index("c")
  sid = lax.axis_index("s")
  wid = sid * NC + cid
  lo = wid * UO
  hi = jnp.minimum(lo + UO, USERS)

  def zrow(j, c):
    for cc in range(D // 16):
      acc[j, pl.ds(cc * 16, 16)] = jnp.zeros((16,), jnp.float32)
    return c
  lax.fori_loop(0, UO, zrow, 0)

  def zmatch(t, c):
    mu[pl.ds(t * 16, 16)] = jnp.full((16,), lo, jnp.int32)
    mv[pl.ds(t * 16, 16)] = jnp.zeros((16,), jnp.float32)
    mp_[pl.ds(t * 16, 16)] = jnp.zeros((16,), jnp.int32)
    return c
  lax.fori_loop(0, FMB // 16, zmatch, 0)

  def fire_strip(st, ub, vb, pb, sem):
    base = st * ST
    pltpu.make_async_copy(u_hbm.at[pl.ds(base, ST)], ub, sem).start()
    pltpu.make_async_copy(val_hbm.at[pl.ds(base, ST)], vb, sem).start()
    pltpu.make_async_copy(p_hbm.at[pl.ds(base, ST)], pb, sem).start()

  def drain_strip(ub, vb, pb, sem):
    pltpu.make_async_copy(u_hbm.at[pl.ds(0, ST)], ub, sem).wait()
    pltpu.make_async_copy(val_hbm.at[pl.ds(0, ST)], vb, sem).wait()
    pltpu.make_async_copy(p_hbm.at[pl.ds(0, ST)], pb, sem).wait()

  def flush(pos):
    nch = (pos + C5 - 1) // C5

    def fire_rows(ch, rbuf, sem):
      pltpu.make_async_copy(xia_hbm.at[mp_.at[pl.ds(ch * C5, C5)]], rbuf,
                            sem).start()

    def zero_tail(ch):
      for g in range(C5 // 16):
        o = ch * C5 + g * 16
        keep = (o + IOTA()) < pos
        mv[pl.ds(o, 16)] = jnp.where(keep, mv[pl.ds(o, 16)], 0.0)

    def do_chunk(ch, rbuf):
      k0 = ch * C5

      gmode = lax.GatherScatterMode.PROMISE_IN_BOUNDS

      def egrp(g, c3):
        o = g * 16
        lu_vec = mu[pl.ds(k0 + o, 16)] - lo
        val_vec = mv[pl.ds(k0 + o, 16)]
        for j in range(16):
          jc = jnp.full((16,), j, jnp.int32)
          row_sp = lu_vec.at[jc].get(mode=gmode)
          val_sp = val_vec.at[jc].get(mode=gmode)
          loads = [rbuf[g * 16 + j, pl.ds(cc * 16, 16)]
                   for cc in range(D // 16)]
          col0 = IOTA()
          for cc in range(D // 16):
            plsc.addupdate_scatter(acc, [row_sp, col0 + (cc * 16)],
                                   loads[cc] * val_sp)
        return c3
      lax.fori_loop(0, C5 // 16, egrp, 0)

    @pl.when(nch > 0)
    def _():
      zero_tail(nch - 1)
      fire_rows(0, rowsA, semRA)

      def pair(t, c):
        c0 = 2 * t
        c1 = 2 * t + 1

        @pl.when(c1 < nch)
        def _():
          fire_rows(c1, rowsB, semRB)
        pltpu.make_async_copy(xia_hbm.at[mp_.at[pl.ds(0, C5)]], rowsA,
                              semRA).wait()
        do_chunk(c0, rowsA)

        @pl.when(c0 + 2 < nch)
        def _():
          fire_rows(c0 + 2, rowsA, semRA)

        @pl.when(c1 < nch)
        def _():
          pltpu.make_async_copy(xia_hbm.at[mp_.at[pl.ds(0, C5)]], rowsB,
                                semRB).wait()
          do_chunk(c1, rowsB)
        return c
      lax.fori_loop(0, (nch + 1) // 2, pair, 0)
    return jnp.int32(0)

  def do_strip(ub, vb, pb, pos0):
    def scan(g, pos):
      o = g * 16
      uv = ub[pl.ds(o, 16)]
      mask = (uv >= lo) & (uv < hi)
      cnt = plsc.all_reduce_population_count(mask)[0]
      plsc.store_compressed(mu.at[pl.ds(pos, 16)], uv, mask=mask)
      plsc.store_compressed(mv.at[pl.ds(pos, 16)], vb[pl.ds(o, 16)],
                            mask=mask)
      plsc.store_compressed(mp_.at[pl.ds(pos, 16)], pb[pl.ds(o, 16)],
                            mask=mask)
      return pos + cnt
    pos = lax.fori_loop(0, ST // 16, scan, pos0)
    return lax.cond(pos > FM - ST, flush, lambda p: p, pos)

  fire_strip(0, uA, vA, pA, semA)

  def strips(t, pos):
    st0 = 2 * t
    fire_strip(st0 + 1, uB, vB, pB, semB)
    drain_strip(uA, vA, pA, semA)
    pos = do_strip(uA, vA, pA, pos)

    @pl.when(st0 + 2 < NST)
    def _():
      fire_strip(st0 + 2, uA, vA, pA, semA)
    drain_strip(uB, vB, pB, semB)
    return do_strip(uB, vB, pB, pos)
  pos = lax.fori_loop(0, NST // 2, strips, jnp.int32(0))
  flush(pos)

  pltpu.sync_copy(acc, nf_hbm.at[pl.ds(lo, UO)])


def _run_agg(up, val, p, xia_flat):
  k = pl.kernel(
      _agg_body,
      out_type=jax.ShapeDtypeStruct((UP, D), jnp.float32),
      mesh=_mesh(),
      compiler_params=_SC_PARAMS,
      scratch_types=[
          pltpu.VMEM((ST,), jnp.int32),
          pltpu.VMEM((ST,), jnp.float32),
          pltpu.VMEM((ST,), jnp.int32),
          pltpu.VMEM((ST,), jnp.int32),
          pltpu.VMEM((ST,), jnp.float32),
          pltpu.VMEM((ST,), jnp.int32),
          pltpu.VMEM((FMB,), jnp.int32),
          pltpu.VMEM((FMB,), jnp.float32),
          pltpu.VMEM((FMB,), jnp.int32),
          pltpu.VMEM((C5, D), jnp.float32),
          pltpu.VMEM((C5, D), jnp.float32),
          pltpu.VMEM((UO, D), jnp.float32),
          pltpu.SemaphoreType.DMA,
          pltpu.SemaphoreType.DMA,
          pltpu.SemaphoreType.DMA,
          pltpu.SemaphoreType.DMA,
      ],
  )
  return k(up, val, p, xia_flat)


def _out_body(nf_ref, w_ref, b_ref, out_ref):
  out_ref[...] = jnp.dot(nf_ref[...], w_ref[...],
                         preferred_element_type=jnp.float32) + b_ref[...]


def _run_out(nf, wT, wb):
  return pl.pallas_call(
      _out_body,
      grid=(5,),
      in_specs=[
          pl.BlockSpec((1000, D), lambda b: (b, 0)),
          pl.BlockSpec((D, D), lambda b: (0, 0)),
          pl.BlockSpec((1, D), lambda b: (0, 0)),
      ],
      out_specs=pl.BlockSpec((1000, D), lambda b: (b, 0)),
      out_shape=jax.ShapeDtypeStruct((USERS, D), jnp.float32),
  )(nf, wT, wb)


@jax.jit
def kernel(i, u, rating, user_feat, item_feat, rating_feat,
           gv_w1, gv_b1, gv_w2, gv_b2,
           att1_w, att1_b, att2_w, att2_b, att3_w, att3_b,
           w_w, w_b):
  i = i.astype(jnp.int32)
  u = u.astype(jnp.int32)
  rating = rating.astype(jnp.int32)

  pad = EP - E
  ip = jnp.concatenate([i, jnp.zeros((pad,), jnp.int32)])
  rp = jnp.concatenate([rating, jnp.zeros((pad,), jnp.int32)])
  up = jnp.concatenate([u, jnp.full((pad,), UP - 1, jnp.int32)])

  rating_pad = jnp.concatenate(
      [rating_feat, jnp.zeros((8 - NR, D), jnp.float32)], axis=0)
  w1iT = gv_w1[:, :D].T
  w1rT = gv_w1[:, D:].T
  b1 = gv_b1.reshape(1, D)
  w2T = gv_w2.T
  b2 = gv_b2.reshape(1, D)
  a1xT = att1_w[:, :D].T
  a1uT = att1_w[:, D:].T
  a1b = att1_b.reshape(1, D)
  a2T = att2_w.T
  a2b = att2_b.reshape(1, D)
  a3 = att3_w.reshape(1, D)
  wT = w_w.T
  wb = w_b.reshape(1, D)

  xia_tab, xa1_tab, up1 = _build_tables(
      item_feat, user_feat, rating_pad, w1iT, w1rT, b1, w2T, b2, a1xT, a1uT,
      a1b)
  xia_flat = xia_tab.reshape(NR * ITEMS, D)
  xa1_flat = xa1_tab.reshape(NR * ITEMS, D)

  g = _run_gather(xa1_flat, up1, ip, rp, up)
  s = _run_score(g, a2T, a2b, a3)
  mp = _run_segmax(s, up)
  ssp = _run_segsum(s, up, mp)
  val, p = _run_val(s, up, ip, rp, mp, ssp)
  nf = _run_agg(up, val, p, xia_flat)
  return _run_out(nf, wT, wb)
